# baseline (speedup 1.0000x reference)
import functools

import jax
import jax.numpy as jnp
from jax import lax
from jax.experimental import pallas as pl
from jax.experimental.pallas import tpu as pltpu
from jax.experimental.pallas import tpu_sc as plsc


def _make_hist(P, L, S):
    info = plsc.get_sparse_core_info()
    nc, ns, nl = info.num_cores, info.num_subcores, info.num_lanes
    nw = nc * ns
    assert P % nw == 0 and L % nl == 0 and S % nl == 0
    ppw = P // nw

    mesh = plsc.VectorSubcoreMesh(core_axis_name="c", subcore_axis_name="s")

    @functools.partial(
        pl.kernel,
        mesh=mesh,
        out_type=jax.ShapeDtypeStruct((P, S), jnp.float32),
        scratch_types=[
            pltpu.VMEM((2, L), jnp.int32),
            pltpu.VMEM((2, nl), jnp.int32),
            pltpu.VMEM((ppw, S), jnp.float32),
            pltpu.SemaphoreType.DMA,
            pltpu.SemaphoreType.DMA,
            pltpu.SemaphoreType.DMA,
            pltpu.SemaphoreType.DMA,
            pltpu.SemaphoreType.DMA,
        ],
        compiler_params=pltpu.CompilerParams(needs_layout_passes=False),
    )
    def hist(ind_hbm, nnzb_hbm, cnt_hbm, idx_v, nnz_v, cnt_v, isem0, isem1,
             nsem0, nsem1, osem):
        wid = lax.axis_index("s") * nc + lax.axis_index("c")
        base = wid * ppw
        ones = jnp.ones((nl,), jnp.float32)
        zeros = jnp.zeros((nl,), jnp.float32)
        lanes = lax.iota(jnp.int32, nl)
        isems = (isem0, isem1)
        nsems = (nsem0, nsem1)

        def in_copies(pp, slot):
            p = base + pp
            return (
                pltpu.make_async_copy(ind_hbm.at[p], idx_v.at[slot], isems[slot]),
                pltpu.make_async_copy(nnzb_hbm.at[p], nnz_v.at[slot], nsems[slot]),
            )

        for c in in_copies(0, 0):
            c.start()

        for r in range(ppw):
            def zero_body(j, carry, r=r):
                cnt_v[r, pl.ds(j * nl, nl)] = zeros
                return carry

            lax.fori_loop(0, S // nl, zero_body, 0, unroll=8)

        for pp in range(ppw):
            slot = pp % 2
            if pp + 1 < ppw:
                for c in in_copies(pp + 1, 1 - slot):
                    c.start()
            for c in in_copies(pp, slot):
                c.wait()
            nnz_vec = nnz_v[slot]
            rows = jnp.full((nl,), pp, jnp.int32)

            def scat_body(i, carry, slot=slot, rows=rows):
                idx = idx_v[slot, pl.ds(i * nl, nl)]
                mask = (lanes + i * nl) < nnz_vec
                plsc.addupdate_scatter(cnt_v, [rows, idx], ones, mask=mask)
                return carry

            lax.fori_loop(0, L // nl, scat_body, 0, unroll=4)
            pltpu.make_async_copy(cnt_v.at[pp], cnt_hbm.at[base + pp], osem).start()

        for pp in range(ppw):
            pltpu.make_async_copy(cnt_v.at[pp], cnt_hbm.at[base + pp], osem).wait()

    return hist


def _attn_body(scale, q_ref, k_ref, v_ref, cnt_ref, o_ref):
    qv = q_ref[0, 0]
    kv = k_ref[0, 0]
    vv = v_ref[0, 0]
    cnt = cnt_ref[0, 0]
    fs = lax.dot_general(qv, kv, (((1,), (1,)), ((), ())),
                         preferred_element_type=jnp.float32) * scale
    m = jnp.max(jnp.where(cnt > 0.0, fs, -jnp.inf), axis=1, keepdims=True)
    w = cnt * jnp.exp(fs - m)
    denom = jnp.sum(w, axis=1, keepdims=True)
    o = lax.dot_general(w, vv, (((1,), (0,)), ((), ())),
                        preferred_element_type=jnp.float32)
    o_ref[0, 0] = o / denom


def _attn(q4, K, V, cnt4, scale):
    B, Hkv, G, D = q4.shape
    S = K.shape[2]
    grid = (B, Hkv)
    blk = lambda s: pl.BlockSpec(s, lambda b, h: (b, h, 0, 0))
    return pl.pallas_call(
        functools.partial(_attn_body, scale),
        grid=grid,
        in_specs=[
            blk((1, 1, G, D)),
            blk((1, 1, S, D)),
            blk((1, 1, S, D)),
            blk((1, 1, G, S)),
        ],
        out_specs=blk((1, 1, G, D)),
        out_shape=jax.ShapeDtypeStruct((B, Hkv, G, D), jnp.float32),
    )(q4, K, V, cnt4)


def kernel(q, K, V, sparse_ind, sparse_nnz, gqa_group_size):
    B, H, _, D = q.shape
    Hkv, S = K.shape[1], K.shape[2]
    G = H // Hkv
    L = sparse_ind.shape[-1]
    P = B * H
    scale = 1.0 / (D ** 0.5)
    info = plsc.get_sparse_core_info()
    nl = info.num_lanes

    ind = sparse_ind.reshape(P, L)
    nnzb = jnp.broadcast_to(
        jnp.maximum(sparse_nnz.astype(jnp.int32), 1).reshape(P, 1), (P, nl)
    )
    q4 = q.reshape(B, Hkv, G, D)
    Bh = B // 2
    Ph = P // 2
    hist = _make_hist(Ph, L, S)
    outs = []
    for i in range(2):
        cnt = hist(ind[i * Ph:(i + 1) * Ph], nnzb[i * Ph:(i + 1) * Ph])
        cnt4 = cnt.reshape(Bh, Hkv, G, S)
        sl = slice(i * Bh, (i + 1) * Bh)
        outs.append(_attn(q4[sl], K[sl], V[sl], cnt4, scale))
    out = jnp.concatenate(outs, axis=0)
    return out.reshape(B, H, 1, D)


# speedup vs baseline: 1.8649x; 1.8649x over previous
import functools

import jax
import jax.numpy as jnp
from jax import lax
from jax.experimental import pallas as pl
from jax.experimental.pallas import tpu as pltpu
from jax.experimental.pallas import tpu_sc as plsc


def _make_hist(P, L, S):
    info = plsc.get_sparse_core_info()
    nc, ns, nl = info.num_cores, info.num_subcores, info.num_lanes
    nw = nc * ns
    assert P % nw == 0 and L % nl == 0 and S % nl == 0
    ppw = P // nw

    mesh = plsc.VectorSubcoreMesh(core_axis_name="c", subcore_axis_name="s")

    @functools.partial(
        pl.kernel,
        mesh=mesh,
        out_type=jax.ShapeDtypeStruct((P, S), jnp.float32),
        scratch_types=[
            pltpu.VMEM((2, L), jnp.int32),
            pltpu.VMEM((2, nl), jnp.int32),
            pltpu.VMEM((ppw, S), jnp.float32),
            pltpu.SemaphoreType.DMA,
            pltpu.SemaphoreType.DMA,
            pltpu.SemaphoreType.DMA,
            pltpu.SemaphoreType.DMA,
            pltpu.SemaphoreType.DMA,
        ],
        compiler_params=pltpu.CompilerParams(needs_layout_passes=False),
    )
    def hist(ind_hbm, nnzb_hbm, cnt_hbm, idx_v, nnz_v, cnt_v, isem0, isem1,
             nsem0, nsem1, osem):
        wid = lax.axis_index("s") * nc + lax.axis_index("c")
        base = wid * ppw
        ones = jnp.ones((nl,), jnp.float32)
        zeros = jnp.zeros((nl,), jnp.float32)
        lanes = lax.iota(jnp.int32, nl)
        isems = (isem0, isem1)
        nsems = (nsem0, nsem1)

        def in_copies(pp, slot):
            p = base + pp
            return (
                pltpu.make_async_copy(ind_hbm.at[p], idx_v.at[slot], isems[slot]),
                pltpu.make_async_copy(nnzb_hbm.at[p], nnz_v.at[slot], nsems[slot]),
            )

        for c in in_copies(0, 0):
            c.start()

        for r in range(ppw):
            def zero_body(j, carry, r=r):
                cnt_v[r, pl.ds(j * nl, nl)] = zeros
                return carry

            lax.fori_loop(0, S // nl, zero_body, 0, unroll=8)

        for pp in range(ppw):
            slot = pp % 2
            if pp + 1 < ppw:
                for c in in_copies(pp + 1, 1 - slot):
                    c.start()
            for c in in_copies(pp, slot):
                c.wait()
            nnz_vec = nnz_v[slot]
            rows = jnp.full((nl,), pp, jnp.int32)

            def scat_body(i, carry, slot=slot, rows=rows):
                idx = idx_v[slot, pl.ds(i * nl, nl)]
                mask = (lanes + i * nl) < nnz_vec
                plsc.addupdate_scatter(cnt_v, [rows, idx], ones, mask=mask)
                return carry

            lax.fori_loop(0, L // nl, scat_body, 0, unroll=4)
            pltpu.make_async_copy(cnt_v.at[pp], cnt_hbm.at[base + pp], osem).start()

        for pp in range(ppw):
            pltpu.make_async_copy(cnt_v.at[pp], cnt_hbm.at[base + pp], osem).wait()

    return hist


def _scores_body(scale, q_ref, k_ref, fs_ref):
    qv = q_ref[0, 0]
    kv = k_ref[0, 0]
    fs_ref[0, 0] = lax.dot_general(qv, kv, (((1,), (1,)), ((), ())),
                                   preferred_element_type=jnp.float32) * scale


def _scores(q4, K, scale):
    B, Hkv, G, D = q4.shape
    S = K.shape[2]
    blk = lambda s: pl.BlockSpec(s, lambda b, h: (b, h, 0, 0))
    return pl.pallas_call(
        functools.partial(_scores_body, scale),
        grid=(B, Hkv),
        in_specs=[blk((1, 1, G, D)), blk((1, 1, S, D))],
        out_specs=blk((1, 1, G, S)),
        out_shape=jax.ShapeDtypeStruct((B, Hkv, G, S), jnp.float32),
    )(q4, K)


def _finish_body(fs_ref, cnt_ref, v_ref, o_ref):
    fs = fs_ref[0, 0]
    cnt = cnt_ref[0, 0]
    vv = v_ref[0, 0]
    m = jnp.max(jnp.where(cnt > 0.0, fs, -jnp.inf), axis=1, keepdims=True)
    w = cnt * jnp.exp(fs - m)
    denom = jnp.sum(w, axis=1, keepdims=True)
    o = lax.dot_general(w, vv, (((1,), (0,)), ((), ())),
                        preferred_element_type=jnp.float32)
    o_ref[0, 0] = o / denom


def _finish(fs, cnt4, V):
    B, Hkv, G, S = fs.shape
    D = V.shape[3]
    blk = lambda s: pl.BlockSpec(s, lambda b, h: (b, h, 0, 0))
    return pl.pallas_call(
        _finish_body,
        grid=(B, Hkv),
        in_specs=[blk((1, 1, G, S)), blk((1, 1, G, S)), blk((1, 1, S, D))],
        out_specs=blk((1, 1, G, D)),
        out_shape=jax.ShapeDtypeStruct((B, Hkv, G, D), jnp.float32),
    )(fs, cnt4, V)


def _attn_body(scale, q_ref, k_ref, v_ref, cnt_ref, o_ref):
    qv = q_ref[0, 0]
    kv = k_ref[0, 0]
    vv = v_ref[0, 0]
    cnt = cnt_ref[0, 0]
    fs = lax.dot_general(qv, kv, (((1,), (1,)), ((), ())),
                         preferred_element_type=jnp.float32) * scale
    m = jnp.max(jnp.where(cnt > 0.0, fs, -jnp.inf), axis=1, keepdims=True)
    w = cnt * jnp.exp(fs - m)
    denom = jnp.sum(w, axis=1, keepdims=True)
    o = lax.dot_general(w, vv, (((1,), (0,)), ((), ())),
                        preferred_element_type=jnp.float32)
    o_ref[0, 0] = o / denom


def _attn(q4, K, V, cnt4, scale):
    B, Hkv, G, D = q4.shape
    S = K.shape[2]
    grid = (B, Hkv)
    blk = lambda s: pl.BlockSpec(s, lambda b, h: (b, h, 0, 0))
    return pl.pallas_call(
        functools.partial(_attn_body, scale),
        grid=grid,
        in_specs=[
            blk((1, 1, G, D)),
            blk((1, 1, S, D)),
            blk((1, 1, S, D)),
            blk((1, 1, G, S)),
        ],
        out_specs=blk((1, 1, G, D)),
        out_shape=jax.ShapeDtypeStruct((B, Hkv, G, D), jnp.float32),
    )(q4, K, V, cnt4)


def kernel(q, K, V, sparse_ind, sparse_nnz, gqa_group_size):
    B, H, _, D = q.shape
    Hkv, S = K.shape[1], K.shape[2]
    G = H // Hkv
    L = sparse_ind.shape[-1]
    P = B * H
    scale = 1.0 / (D ** 0.5)
    info = plsc.get_sparse_core_info()
    nl = info.num_lanes

    ind = sparse_ind.reshape(P, L)
    nnzb = jnp.broadcast_to(
        jnp.maximum(sparse_nnz.astype(jnp.int32), 1).reshape(P, 1), (P, nl)
    )
    q4 = q.reshape(B, Hkv, G, D)
    cnt = _make_hist(P, L, S)(ind, nnzb)
    fs = _scores(q4, K, scale)
    cnt4 = cnt.reshape(B, Hkv, G, S)
    out = _finish(fs, cnt4, V)
    return out.reshape(B, H, 1, D)


# speedup vs baseline: 2.1824x; 1.1702x over previous
import functools

import jax
import jax.numpy as jnp
from jax import lax
from jax.experimental import pallas as pl
from jax.experimental.pallas import tpu as pltpu
from jax.experimental.pallas import tpu_sc as plsc


def _make_hist(P, L, S):
    info = plsc.get_sparse_core_info()
    nc, ns, nl = info.num_cores, info.num_subcores, info.num_lanes
    nw = nc * ns
    assert P % nw == 0 and L % nl == 0 and S % nl == 0
    ppw = P // nw

    mesh = plsc.VectorSubcoreMesh(core_axis_name="c", subcore_axis_name="s")

    @functools.partial(
        pl.kernel,
        mesh=mesh,
        out_type=jax.ShapeDtypeStruct((P, S), jnp.float32),
        scratch_types=[
            pltpu.VMEM((2, L), jnp.int32),
            pltpu.VMEM((2, nl), jnp.int32),
            pltpu.VMEM((ppw, S), jnp.float32),
            pltpu.SemaphoreType.DMA,
            pltpu.SemaphoreType.DMA,
            pltpu.SemaphoreType.DMA,
            pltpu.SemaphoreType.DMA,
            pltpu.SemaphoreType.DMA,
            pltpu.SemaphoreType.DMA,
        ],
        compiler_params=pltpu.CompilerParams(needs_layout_passes=False),
    )
    def hist(ind_hbm, nnzb_hbm, zeros_hbm, cnt_hbm, idx_v, nnz_v, cnt_v,
             isem0, isem1, nsem0, nsem1, osem, zsem):
        wid = lax.axis_index("s") * nc + lax.axis_index("c")
        base = wid * ppw
        ones = jnp.ones((nl,), jnp.float32)
        lanes = lax.iota(jnp.int32, nl)
        isems = (isem0, isem1)
        nsems = (nsem0, nsem1)

        def in_copies(pp, slot):
            p = base + pp
            return (
                pltpu.make_async_copy(ind_hbm.at[p], idx_v.at[slot], isems[slot]),
                pltpu.make_async_copy(nnzb_hbm.at[p], nnz_v.at[slot], nsems[slot]),
            )

        zcopy = pltpu.make_async_copy(zeros_hbm, cnt_v, zsem)
        zcopy.start()
        for c in in_copies(0, 0):
            c.start()
        zcopy.wait()

        for pp in range(ppw):
            slot = pp % 2
            if pp + 1 < ppw:
                for c in in_copies(pp + 1, 1 - slot):
                    c.start()
            for c in in_copies(pp, slot):
                c.wait()
            nnz_vec = nnz_v[slot]
            rows = jnp.full((nl,), pp, jnp.int32)

            def scat_body(i, carry, slot=slot, rows=rows):
                idx = idx_v[slot, pl.ds(i * nl, nl)]
                mask = (lanes + i * nl) < nnz_vec
                plsc.addupdate_scatter(cnt_v, [rows, idx], ones, mask=mask)
                return carry

            lax.fori_loop(0, L // nl, scat_body, 0, unroll=4)
            pltpu.make_async_copy(cnt_v.at[pp], cnt_hbm.at[base + pp], osem).start()

        for pp in range(ppw):
            pltpu.make_async_copy(cnt_v.at[pp], cnt_hbm.at[base + pp], osem).wait()

    return hist


def _attn_body(scale, q_ref, k_ref, v_ref, cnt_ref, o_ref):
    qv = q_ref[0, 0]
    kv = k_ref[0, 0]
    vv = v_ref[0, 0]
    cnt = cnt_ref[0, 0]
    fs = lax.dot_general(qv, kv, (((1,), (1,)), ((), ())),
                         preferred_element_type=jnp.float32) * scale
    m = jnp.max(jnp.where(cnt > 0.0, fs, -jnp.inf), axis=1, keepdims=True)
    w = cnt * jnp.exp(fs - m)
    denom = jnp.sum(w, axis=1, keepdims=True)
    o = lax.dot_general(w, vv, (((1,), (0,)), ((), ())),
                        preferred_element_type=jnp.float32)
    o_ref[0, 0] = o / denom


def _attn(q4, K, V, cnt4, scale):
    B, Hkv, G, D = q4.shape
    S = K.shape[2]
    grid = (B, Hkv)
    blk = lambda s: pl.BlockSpec(s, lambda b, h: (b, h, 0, 0))
    return pl.pallas_call(
        functools.partial(_attn_body, scale),
        grid=grid,
        in_specs=[
            blk((1, 1, G, D)),
            blk((1, 1, S, D)),
            blk((1, 1, S, D)),
            blk((1, 1, G, S)),
        ],
        out_specs=blk((1, 1, G, D)),
        out_shape=jax.ShapeDtypeStruct((B, Hkv, G, D), jnp.float32),
    )(q4, K, V, cnt4)


def kernel(q, K, V, sparse_ind, sparse_nnz, gqa_group_size):
    B, H, _, D = q.shape
    Hkv, S = K.shape[1], K.shape[2]
    G = H // Hkv
    L = sparse_ind.shape[-1]
    P = B * H
    scale = 1.0 / (D ** 0.5)
    info = plsc.get_sparse_core_info()
    nl = info.num_lanes

    ind = sparse_ind.reshape(P, L)
    nnzb = jnp.broadcast_to(
        jnp.maximum(sparse_nnz.astype(jnp.int32), 1).reshape(P, 1), (P, nl)
    )
    q4 = q.reshape(B, Hkv, G, D)
    nw = info.num_cores * info.num_subcores
    zeros = jnp.zeros((P // nw, S), jnp.float32)
    cnt = _make_hist(P, L, S)(ind, nnzb, zeros)
    cnt4 = cnt.reshape(B, Hkv, G, S)
    out = _attn(q4, K, V, cnt4, scale)
    return out.reshape(B, H, 1, D)


# speedup vs baseline: 2.2149x; 1.0149x over previous
import functools

import jax
import jax.numpy as jnp
from jax import lax
from jax.experimental import pallas as pl
from jax.experimental.pallas import tpu as pltpu
from jax.experimental.pallas import tpu_sc as plsc


def _make_hist(P, L, S):
    info = plsc.get_sparse_core_info()
    nc, ns, nl = info.num_cores, info.num_subcores, info.num_lanes
    nw = nc * ns
    assert P % nw == 0 and L % nl == 0 and S % nl == 0
    ppw = P // nw

    mesh = plsc.VectorSubcoreMesh(core_axis_name="c", subcore_axis_name="s")

    @functools.partial(
        pl.kernel,
        mesh=mesh,
        out_type=jax.ShapeDtypeStruct((P, S), jnp.float32),
        scratch_types=[
            pltpu.VMEM((2, L), jnp.int32),
            pltpu.VMEM((2, nl), jnp.int32),
            pltpu.VMEM((ppw, S), jnp.float32),
            pltpu.SemaphoreType.DMA,
            pltpu.SemaphoreType.DMA,
            pltpu.SemaphoreType.DMA,
            pltpu.SemaphoreType.DMA,
            pltpu.SemaphoreType.DMA,
        ],
        compiler_params=pltpu.CompilerParams(needs_layout_passes=False),
    )
    def hist(ind_hbm, nnzb_hbm, cnt_hbm, idx_v, nnz_v, cnt_v,
             isem0, isem1, nsem0, nsem1, osem):
        wid = lax.axis_index("s") * nc + lax.axis_index("c")
        base = wid * ppw
        ones = jnp.ones((nl,), jnp.float32)
        lanes = lax.iota(jnp.int32, nl)
        isems = (isem0, isem1)
        nsems = (nsem0, nsem1)

        def in_copies(pp, slot):
            p = base + pp
            return (
                pltpu.make_async_copy(ind_hbm.at[p], idx_v.at[slot], isems[slot]),
                pltpu.make_async_copy(nnzb_hbm.at[p], nnz_v.at[slot], nsems[slot]),
            )

        for c in in_copies(0, 0):
            c.start()

        zeros = jnp.zeros((nl,), jnp.float32)
        for r in range(ppw):
            def zero_body(j, carry, r=r):
                cnt_v[r, pl.ds(j * nl, nl)] = zeros
                return carry

            lax.fori_loop(0, S // nl, zero_body, 0, unroll=16)

        for pp in range(ppw):
            slot = pp % 2
            if pp + 1 < ppw:
                for c in in_copies(pp + 1, 1 - slot):
                    c.start()
            for c in in_copies(pp, slot):
                c.wait()
            nnz_vec = nnz_v[slot]
            rows = jnp.full((nl,), pp, jnp.int32)

            def scat_body(i, carry, slot=slot, rows=rows):
                idx = idx_v[slot, pl.ds(i * nl, nl)]
                mask = (lanes + i * nl) < nnz_vec
                plsc.addupdate_scatter(cnt_v, [rows, idx], ones, mask=mask)
                return carry

            lax.fori_loop(0, L // nl, scat_body, 0, unroll=8)
            pltpu.make_async_copy(cnt_v.at[pp], cnt_hbm.at[base + pp], osem).start()

        for pp in range(ppw):
            pltpu.make_async_copy(cnt_v.at[pp], cnt_hbm.at[base + pp], osem).wait()

    return hist


def _attn_body(scale, q_ref, k_ref, v_ref, cnt_ref, o_ref):
    qv = q_ref[0, 0]
    kv = k_ref[0, 0]
    vv = v_ref[0, 0]
    cnt = cnt_ref[0, 0]
    fs = lax.dot_general(qv, kv, (((1,), (1,)), ((), ())),
                         preferred_element_type=jnp.float32) * scale
    m = jnp.max(jnp.where(cnt > 0.0, fs, -jnp.inf), axis=1, keepdims=True)
    w = cnt * jnp.exp(fs - m)
    denom = jnp.sum(w, axis=1, keepdims=True)
    o = lax.dot_general(w, vv, (((1,), (0,)), ((), ())),
                        preferred_element_type=jnp.float32)
    o_ref[0, 0] = o / denom


def _attn(q4, K, V, cnt4, scale):
    B, Hkv, G, D = q4.shape
    S = K.shape[2]
    grid = (B, Hkv)
    blk = lambda s: pl.BlockSpec(s, lambda b, h: (b, h, 0, 0))
    return pl.pallas_call(
        functools.partial(_attn_body, scale),
        grid=grid,
        in_specs=[
            blk((1, 1, G, D)),
            blk((1, 1, S, D)),
            blk((1, 1, S, D)),
            blk((1, 1, G, S)),
        ],
        out_specs=blk((1, 1, G, D)),
        out_shape=jax.ShapeDtypeStruct((B, Hkv, G, D), jnp.float32),
    )(q4, K, V, cnt4)


def kernel(q, K, V, sparse_ind, sparse_nnz, gqa_group_size):
    B, H, _, D = q.shape
    Hkv, S = K.shape[1], K.shape[2]
    G = H // Hkv
    L = sparse_ind.shape[-1]
    P = B * H
    scale = 1.0 / (D ** 0.5)
    info = plsc.get_sparse_core_info()
    nl = info.num_lanes

    ind = sparse_ind.reshape(P, L)
    nnzb = jnp.broadcast_to(
        jnp.maximum(sparse_nnz.astype(jnp.int32), 1).reshape(P, 1), (P, nl)
    )
    q4 = q.reshape(B, Hkv, G, D)
    cnt = _make_hist(P, L, S)(ind, nnzb)
    cnt4 = cnt.reshape(B, Hkv, G, S)
    out = _attn(q4, K, V, cnt4, scale)
    return out.reshape(B, H, 1, D)


# speedup vs baseline: 2.5281x; 1.1414x over previous
import functools

import jax
import jax.numpy as jnp
from jax import lax
from jax.experimental import pallas as pl
from jax.experimental.pallas import tpu as pltpu
from jax.experimental.pallas import tpu_sc as plsc


def _make_hist(P, L, S):
    info = plsc.get_sparse_core_info()
    nc, ns, nl = info.num_cores, info.num_subcores, info.num_lanes
    nw = nc * ns
    assert P % nw == 0 and L % nl == 0 and S % nl == 0
    ppw = P // nw

    mesh = plsc.VectorSubcoreMesh(core_axis_name="c", subcore_axis_name="s")

    @functools.partial(
        pl.kernel,
        mesh=mesh,
        out_type=jax.ShapeDtypeStruct((P, S), jnp.float32),
        scratch_types=[
            pltpu.VMEM((2, L), jnp.int32),
            pltpu.VMEM((2, nl), jnp.int32),
            pltpu.VMEM((ppw, S), jnp.float32),
            pltpu.SemaphoreType.DMA,
            pltpu.SemaphoreType.DMA,
            pltpu.SemaphoreType.DMA,
            pltpu.SemaphoreType.DMA,
            pltpu.SemaphoreType.DMA,
        ],
        compiler_params=pltpu.CompilerParams(needs_layout_passes=False),
    )
    def hist(ind_hbm, nnzb_hbm, cnt_hbm, idx_v, nnz_v, cnt_v,
             isem0, isem1, nsem0, nsem1, osem):
        wid = lax.axis_index("s") * nc + lax.axis_index("c")
        base = wid * ppw
        ones = jnp.ones((nl,), jnp.float32)
        lanes = lax.iota(jnp.int32, nl)
        isems = (isem0, isem1)
        nsems = (nsem0, nsem1)

        def in_copies(pp, slot):
            p = base + pp
            return (
                pltpu.make_async_copy(ind_hbm.at[p], idx_v.at[slot], isems[slot]),
                pltpu.make_async_copy(nnzb_hbm.at[p], nnz_v.at[slot], nsems[slot]),
            )

        for c in in_copies(0, 0):
            c.start()

        zeros = jnp.zeros((nl,), jnp.float32)
        for r in range(ppw):
            def zero_body(j, carry, r=r):
                cnt_v[r, pl.ds(j * nl, nl)] = zeros
                return carry

            lax.fori_loop(0, S // nl, zero_body, 0, unroll=16)

        for pp in range(ppw):
            slot = pp % 2
            if pp + 1 < ppw:
                for c in in_copies(pp + 1, 1 - slot):
                    c.start()
            for c in in_copies(pp, slot):
                c.wait()
            nnz_vec = nnz_v[slot]
            rows = jnp.full((nl,), pp, jnp.int32)

            def scat_body(i, carry, slot=slot, rows=rows):
                idx = idx_v[slot, pl.ds(i * nl, nl)]
                mask = (lanes + i * nl) < nnz_vec
                plsc.addupdate_scatter(cnt_v, [rows, idx], ones, mask=mask)
                return carry

            lax.fori_loop(0, L // nl, scat_body, 0, unroll=8)
            pltpu.make_async_copy(cnt_v.at[pp], cnt_hbm.at[base + pp], osem).start()

        for pp in range(ppw):
            pltpu.make_async_copy(cnt_v.at[pp], cnt_hbm.at[base + pp], osem).wait()

    return hist


def _attn_body(scale, hpb, q_ref, k_ref, v_ref, cnt_ref, o_ref):
    for j in range(hpb):
        qv = q_ref[0, j]
        kv = k_ref[0, j]
        vv = v_ref[0, j]
        cnt = cnt_ref[0, j]
        fs = lax.dot_general(qv, kv, (((1,), (1,)), ((), ())),
                             preferred_element_type=jnp.float32) * scale
        m = jnp.max(jnp.where(cnt > 0.0, fs, -jnp.inf), axis=1, keepdims=True)
        w = cnt * jnp.exp(fs - m)
        denom = jnp.sum(w, axis=1, keepdims=True)
        o = lax.dot_general(w, vv, (((1,), (0,)), ((), ())),
                            preferred_element_type=jnp.float32)
        o_ref[0, j] = o / denom


def _attn(q4, K, V, cnt4, scale, hpb=2):
    B, Hkv, G, D = q4.shape
    S = K.shape[2]
    grid = (B, Hkv // hpb)
    blk = lambda s: pl.BlockSpec(s, lambda b, h: (b, h, 0, 0))
    return pl.pallas_call(
        functools.partial(_attn_body, scale, hpb),
        grid=grid,
        in_specs=[
            blk((1, hpb, G, D)),
            blk((1, hpb, S, D)),
            blk((1, hpb, S, D)),
            blk((1, hpb, G, S)),
        ],
        out_specs=blk((1, hpb, G, D)),
        out_shape=jax.ShapeDtypeStruct((B, Hkv, G, D), jnp.float32),
    )(q4, K, V, cnt4)


def kernel(q, K, V, sparse_ind, sparse_nnz, gqa_group_size):
    B, H, _, D = q.shape
    Hkv, S = K.shape[1], K.shape[2]
    G = H // Hkv
    L = sparse_ind.shape[-1]
    P = B * H
    scale = 1.0 / (D ** 0.5)
    info = plsc.get_sparse_core_info()
    nl = info.num_lanes

    ind = sparse_ind.reshape(P, L)
    nnzb = jnp.broadcast_to(
        jnp.maximum(sparse_nnz.astype(jnp.int32), 1).reshape(P, 1), (P, nl)
    )
    q4 = q.reshape(B, Hkv, G, D)
    cnt = _make_hist(P, L, S)(ind, nnzb)
    cnt4 = cnt.reshape(B, Hkv, G, S)
    out = _attn(q4, K, V, cnt4, scale)
    return out.reshape(B, H, 1, D)


# speedup vs baseline: 2.5395x; 1.0045x over previous
import functools

import jax
import jax.numpy as jnp
from jax import lax
from jax.experimental import pallas as pl
from jax.experimental.pallas import tpu as pltpu
from jax.experimental.pallas import tpu_sc as plsc


def _make_hist(P, L, S):
    info = plsc.get_sparse_core_info()
    nc, ns, nl = info.num_cores, info.num_subcores, info.num_lanes
    nw = nc * ns
    assert P % nw == 0 and L % nl == 0 and S % nl == 0
    ppw = P // nw

    mesh = plsc.VectorSubcoreMesh(core_axis_name="c", subcore_axis_name="s")

    @functools.partial(
        pl.kernel,
        mesh=mesh,
        out_type=jax.ShapeDtypeStruct((P, S), jnp.float32),
        scratch_types=[
            pltpu.VMEM((2, L), jnp.int32),
            pltpu.VMEM((2, nl), jnp.int32),
            pltpu.VMEM((ppw, S), jnp.float32),
            pltpu.SemaphoreType.DMA,
            pltpu.SemaphoreType.DMA,
            pltpu.SemaphoreType.DMA,
            pltpu.SemaphoreType.DMA,
            pltpu.SemaphoreType.DMA,
        ],
        compiler_params=pltpu.CompilerParams(needs_layout_passes=False),
    )
    def hist(ind_hbm, nnzb_hbm, cnt_hbm, idx_v, nnz_v, cnt_v,
             isem0, isem1, nsem0, nsem1, osem):
        wid = lax.axis_index("s") * nc + lax.axis_index("c")
        base = wid * ppw
        ones = jnp.ones((nl,), jnp.float32)
        lanes = lax.iota(jnp.int32, nl)
        isems = (isem0, isem1)
        nsems = (nsem0, nsem1)

        def in_copies(pp, slot):
            p = base + pp
            return (
                pltpu.make_async_copy(ind_hbm.at[p], idx_v.at[slot], isems[slot]),
                pltpu.make_async_copy(nnzb_hbm.at[p], nnz_v.at[slot], nsems[slot]),
            )

        for c in in_copies(0, 0):
            c.start()

        zeros = jnp.zeros((nl,), jnp.float32)
        for r in range(ppw):
            def zero_body(j, carry, r=r):
                cnt_v[r, pl.ds(j * nl, nl)] = zeros
                return carry

            lax.fori_loop(0, S // nl, zero_body, 0, unroll=16)

        for pp in range(ppw):
            slot = pp % 2
            if pp + 1 < ppw:
                for c in in_copies(pp + 1, 1 - slot):
                    c.start()
            for c in in_copies(pp, slot):
                c.wait()
            nnz_vec = nnz_v[slot]
            rows = jnp.full((nl,), pp, jnp.int32)

            def scat_body(i, carry, slot=slot, rows=rows):
                idx = idx_v[slot, pl.ds(i * nl, nl)]
                mask = (lanes + i * nl) < nnz_vec
                plsc.addupdate_scatter(cnt_v, [rows, idx], ones, mask=mask)
                return carry

            lax.fori_loop(0, L // nl, scat_body, 0, unroll=8)
            pltpu.make_async_copy(cnt_v.at[pp], cnt_hbm.at[base + pp], osem).start()

        for pp in range(ppw):
            pltpu.make_async_copy(cnt_v.at[pp], cnt_hbm.at[base + pp], osem).wait()

    return hist


def _attn_body(scale, hpb, q_ref, k_ref, v_ref, cnt_ref, o_ref):
    for j in range(hpb):
        qv = q_ref[0, j]
        kv = k_ref[0, j]
        vv = v_ref[0, j]
        cnt = cnt_ref[0, j]
        fs = lax.dot_general(qv, kv, (((1,), (1,)), ((), ())),
                             preferred_element_type=jnp.float32) * scale
        m = jnp.max(jnp.where(cnt > 0.0, fs, -jnp.inf), axis=1, keepdims=True)
        w = cnt * jnp.exp(fs - m)
        denom = jnp.sum(w, axis=1, keepdims=True)
        o = lax.dot_general(w, vv, (((1,), (0,)), ((), ())),
                            preferred_element_type=jnp.float32)
        o_ref[0, j] = o / denom


def _attn(q4, K, V, cnt4, scale, hpb=4):
    B, Hkv, G, D = q4.shape
    S = K.shape[2]
    grid = (B, Hkv // hpb)
    blk = lambda s: pl.BlockSpec(s, lambda b, h: (b, h, 0, 0))
    return pl.pallas_call(
        functools.partial(_attn_body, scale, hpb),
        grid=grid,
        in_specs=[
            blk((1, hpb, G, D)),
            blk((1, hpb, S, D)),
            blk((1, hpb, S, D)),
            blk((1, hpb, G, S)),
        ],
        out_specs=blk((1, hpb, G, D)),
        out_shape=jax.ShapeDtypeStruct((B, Hkv, G, D), jnp.float32),
    )(q4, K, V, cnt4)


def kernel(q, K, V, sparse_ind, sparse_nnz, gqa_group_size):
    B, H, _, D = q.shape
    Hkv, S = K.shape[1], K.shape[2]
    G = H // Hkv
    L = sparse_ind.shape[-1]
    P = B * H
    scale = 1.0 / (D ** 0.5)
    info = plsc.get_sparse_core_info()
    nl = info.num_lanes

    ind = sparse_ind.reshape(P, L)
    nnzb = jnp.broadcast_to(
        jnp.maximum(sparse_nnz.astype(jnp.int32), 1).reshape(P, 1), (P, nl)
    )
    q4 = q.reshape(B, Hkv, G, D)
    cnt = _make_hist(P, L, S)(ind, nnzb)
    cnt4 = cnt.reshape(B, Hkv, G, S)
    out = _attn(q4, K, V, cnt4, scale)
    return out.reshape(B, H, 1, D)
